# Initial kernel scaffold; baseline (speedup 1.0000x reference)
import jax
import jax.numpy as jnp
from jax.experimental import pallas as pl


def kernel(entity, node_type, edge_index, entity_emb, type_emb, W1, b1, W2, b2, Wc, bc):
    raise NotImplementedError("write your pallas kernel here")


# speedup vs baseline: 13.5345x; 13.5345x over previous
import functools

import jax
import jax.numpy as jnp
from jax import lax
from jax.experimental import pallas as pl
from jax.experimental.pallas import tpu as pltpu
from jax.experimental.pallas import tpu_sc as plsc

N = 10000
E = 320000
D = 128
NB = 79
NP = NB * 128
NPAD = NP - N
NC, NS = 2, 16
NW = NC * NS
ECH = 79
EPT = ECH * 128
EP = NW * EPT

_MESH = plsc.VectorSubcoreMesh(
    core_axis_name="c", subcore_axis_name="s", num_cores=NC, num_subcores=NS)


@functools.partial(
    pl.kernel,
    out_type=(
        jax.ShapeDtypeStruct((NP, D), jnp.float32),
        jax.ShapeDtypeStruct((NW, NP), jnp.float32),
    ),
    mesh=_MESH,
    scratch_types=[
        pltpu.VMEM((128,), jnp.int32),
        pltpu.VMEM((128, D), jnp.float32),
        pltpu.VMEM((EPT,), jnp.int32),
        pltpu.VMEM((NP,), jnp.float32),
        pltpu.SemaphoreType.DMA,
    ],
    compiler_params=pltpu.CompilerParams(needs_layout_passes=False),
)
def _stage0(table_hbm, ent_hbm, dst_hbm, zcol_hbm,
            xg_hbm, degp_hbm,
            idx_v, rows_v, ddst_v, deg_v, sem):
    cid = lax.axis_index("c")
    sid = lax.axis_index("s")
    wid = sid * NC + cid

    for k in range(3):
        b = wid + k * NW

        @pl.when(b < NB)
        def _():
            pltpu.sync_copy(ent_hbm.at[pl.ds(b * 128, 128)], idx_v)
            pltpu.async_copy(table_hbm.at[idx_v], rows_v, sem).wait()
            pltpu.sync_copy(rows_v, xg_hbm.at[pl.ds(b * 128, 128), :])

    pltpu.sync_copy(zcol_hbm, deg_v)
    pltpu.sync_copy(dst_hbm.at[pl.ds(wid * EPT, EPT)], ddst_v)
    ones16 = jnp.full((16,), 1.0, jnp.float32)

    def body(i, carry):
        v = ddst_v[pl.ds(i * 16, 16)]
        plsc.addupdate_scatter(deg_v, [v], ones16)
        return carry

    lax.fori_loop(0, EPT // 16, body, 0)
    pltpu.sync_copy(deg_v, degp_hbm.at[wid])


@functools.partial(
    pl.kernel,
    out_type=jax.ShapeDtypeStruct((NC, NP, D), jnp.float32),
    mesh=_MESH,
    scratch_types=[
        pltpu.VMEM_SHARED((NP, D), jnp.float32),
        pltpu.VMEM((ECH, 128), jnp.int32),
        pltpu.VMEM((ECH, 128), jnp.int32),
        pltpu.VMEM((128, D), jnp.float32),
        pltpu.SemaphoreType.DMA,
    ],
)
def _msgpass(hs_hbm, src_hbm, dst_hbm, zeros_hbm,
             aggp_hbm,
             acc_sh, sidx_v, didx_v, rows_v, sem):
    cid = lax.axis_index("c")
    sid = lax.axis_index("s")
    wid = sid * NC + cid

    @pl.when(sid == 0)
    def _():
        pltpu.sync_copy(zeros_hbm, acc_sh)

    pltpu.sync_copy(src_hbm.at[wid], sidx_v)
    pltpu.sync_copy(dst_hbm.at[wid], didx_v)
    plsc.subcore_barrier()

    def body(k, carry):
        pltpu.async_copy(hs_hbm.at[sidx_v.at[k]], rows_v, sem).wait()
        pltpu.sync_copy(rows_v, acc_sh.at[didx_v.at[k]], add=True)
        return carry

    lax.fori_loop(0, ECH, body, 0)
    plsc.subcore_barrier()

    for k in range(5):
        b = sid + k * NS

        @pl.when(b < NB)
        def _():
            pltpu.sync_copy(acc_sh.at[pl.ds(b * 128, 128)],
                            aggp_hbm.at[cid, pl.ds(b * 128, 128)])


def _tc_a_body(xg_ref, nt_ref, te_ref, degp_ref, w1_ref, hs_ref, dinv_ref):
    nt = nt_ref[...]
    te = te_ref[...]
    x = xg_ref[...] + jnp.where(nt == 0, te[0:1],
                                jnp.where(nt == 1, te[1:2], te[2:3]))
    deg = jnp.sum(degp_ref[...], axis=0) + 1.0
    dinv = lax.rsqrt(deg)
    h = lax.dot_general(x, w1_ref[...], (((1,), (0,)), ((), ())),
                        precision=lax.Precision.HIGHEST)
    hs_ref[...] = dinv * h
    dinv_ref[...] = dinv


def _tc_c_body(aggp_ref, hs_ref, dinv_ref, b1_ref, w2_ref, out_ref):
    dinv = dinv_ref[...]
    a = aggp_ref[0] + aggp_ref[1] + hs_ref[...]
    z = jnp.maximum(dinv * a + b1_ref[...], 0.0)
    h2 = lax.dot_general(z, w2_ref[...], (((1,), (0,)), ((), ())),
                         precision=lax.Precision.HIGHEST)
    out_ref[...] = dinv * h2


def _tc_e_body(aggp_ref, hs_ref, dinv_ref, b2_ref, wc_ref, bc_ref, out_ref):
    a = aggp_ref[0] + aggp_ref[1] + hs_ref[...]
    z = dinv_ref[...] * a + b2_ref[...]
    out_ref[...] = lax.dot_general(z, wc_ref[...], (((1,), (0,)), ((), ())),
                                   precision=lax.Precision.HIGHEST) + bc_ref[...]


_row_spec = pl.BlockSpec((128, D), lambda i: (i, 0))
_col_spec = pl.BlockSpec((128, 1), lambda i: (i, 0))
_full_w = pl.BlockSpec((D, D), lambda i: (0, 0))
_full_b = pl.BlockSpec((1, D), lambda i: (0, 0))
_aggp_spec = pl.BlockSpec((NC, 128, D), lambda i: (0, i, 0))

_tc_a = pl.pallas_call(
    _tc_a_body,
    grid=(NB,),
    in_specs=[
        _row_spec,
        pl.BlockSpec((128, 1), lambda i: (i, 0)),
        pl.BlockSpec((8, D), lambda i: (0, 0)),
        pl.BlockSpec((NW, 128, 1), lambda i: (0, i, 0)),
        _full_w,
    ],
    out_specs=[_row_spec, _col_spec],
    out_shape=[
        jax.ShapeDtypeStruct((NP, D), jnp.float32),
        jax.ShapeDtypeStruct((NP, 1), jnp.float32),
    ],
)

_tc_c = pl.pallas_call(
    _tc_c_body,
    grid=(NB,),
    in_specs=[_aggp_spec, _row_spec, _col_spec, _full_b, _full_w],
    out_specs=_row_spec,
    out_shape=jax.ShapeDtypeStruct((NP, D), jnp.float32),
)

_tc_e = pl.pallas_call(
    _tc_e_body,
    grid=(NB,),
    in_specs=[_aggp_spec, _row_spec, _col_spec, _full_b, _full_w, _full_b],
    out_specs=_row_spec,
    out_shape=jax.ShapeDtypeStruct((NP, D), jnp.float32),
)


def kernel(entity, node_type, edge_index, entity_emb, type_emb,
           W1, b1, W2, b2, Wc, bc):
    i32 = jnp.int32
    ent_p = jnp.concatenate(
        [entity.astype(i32), jnp.zeros((NPAD,), i32)])
    nt_col = jnp.concatenate(
        [node_type.astype(i32), jnp.zeros((NPAD,), i32)]).reshape(NP, 1)
    pad_idx = (jnp.arange(EP - E, dtype=i32) % NPAD) + N
    src_p = jnp.concatenate([edge_index[0].astype(i32), pad_idx])
    dst_p = jnp.concatenate([edge_index[1].astype(i32), pad_idx])
    src3 = src_p.reshape(NW, ECH, 128)
    dst3 = dst_p.reshape(NW, ECH, 128)

    te_p = jnp.zeros((8, D), jnp.float32).at[:3].set(type_emb)
    wc_p = jnp.zeros((D, D), jnp.float32).at[:, :3].set(Wc)
    bc_p = jnp.zeros((1, D), jnp.float32).at[0, :3].set(bc)
    zcol = jnp.zeros((NP,), jnp.float32)
    zbig = jnp.zeros((NP, D), jnp.float32)

    xg, degp = _stage0(entity_emb, ent_p, dst_p, zcol)
    hs1, dinv = _tc_a(xg, nt_col, te_p, degp.reshape(NW, NP, 1), W1)
    agg1 = _msgpass(hs1, src3, dst3, zbig)
    hs2 = _tc_c(agg1, hs1, dinv, b1.reshape(1, D), W2)
    agg2 = _msgpass(hs2, src3, dst3, zbig)
    logits_p = _tc_e(agg2, hs2, dinv, b2.reshape(1, D), wc_p, bc_p)
    return logits_p[:N, :3]


# speedup vs baseline: 20.4455x; 1.5106x over previous
import functools

import jax
import jax.numpy as jnp
from jax import lax
from jax.experimental import pallas as pl
from jax.experimental.pallas import tpu as pltpu
from jax.experimental.pallas import tpu_sc as plsc

N = 10000
E = 320000
D = 128
NB = 79
NP = NB * 128
NPAD = NP - N
NC, NS = 2, 16
NW = NC * NS
ECH = 79
EPT = ECH * 128
EP = NW * EPT

_MESH = plsc.VectorSubcoreMesh(
    core_axis_name="c", subcore_axis_name="s", num_cores=NC, num_subcores=NS)


@functools.partial(
    pl.kernel,
    out_type=(
        jax.ShapeDtypeStruct((NP, D), jnp.float32),
        jax.ShapeDtypeStruct((NW, NB, 128), jnp.float32),
    ),
    mesh=_MESH,
    scratch_types=[
        pltpu.VMEM((128,), jnp.int32),
        pltpu.VMEM((128, D), jnp.float32),
        pltpu.VMEM((EPT,), jnp.int32),
        pltpu.VMEM((NB, 128), jnp.float32),
        pltpu.SemaphoreType.DMA,
    ],
    compiler_params=pltpu.CompilerParams(needs_layout_passes=False),
)
def _stage0(table_hbm, ent_hbm, dst_hbm, zcol_hbm,
            xg_hbm, degp_hbm,
            idx_v, rows_v, ddst_v, deg_v, sem):
    cid = lax.axis_index("c")
    sid = lax.axis_index("s")
    wid = sid * NC + cid

    for k in range(3):
        b = wid + k * NW

        @pl.when(b < NB)
        def _():
            pltpu.sync_copy(ent_hbm.at[pl.ds(b * 128, 128)], idx_v)
            pltpu.async_copy(table_hbm.at[idx_v], rows_v, sem).wait()
            pltpu.sync_copy(rows_v, xg_hbm.at[pl.ds(b * 128, 128), :])

    pltpu.sync_copy(zcol_hbm, deg_v)
    pltpu.sync_copy(dst_hbm.at[pl.ds(wid * EPT, EPT)], ddst_v)
    ones16 = jnp.full((16,), 1.0, jnp.float32)

    def body(i, carry):
        v = ddst_v[pl.ds(i * 16, 16)]
        plsc.addupdate_scatter(
            deg_v, [lax.shift_right_logical(v, 7), lax.bitwise_and(v, 127)],
            ones16)
        return carry

    lax.fori_loop(0, EPT // 16, body, 0)
    pltpu.sync_copy(deg_v, degp_hbm.at[wid])


@functools.partial(
    pl.kernel,
    out_type=jax.ShapeDtypeStruct((NC, NP, D), jnp.float32),
    mesh=_MESH,
    scratch_types=[
        pltpu.VMEM_SHARED((NP, D), jnp.float32),
        pltpu.VMEM((ECH, 128), jnp.int32),
        pltpu.VMEM((ECH, 128), jnp.int32),
        pltpu.VMEM((128, D), jnp.float32),
        pltpu.SemaphoreType.DMA,
    ],
)
def _msgpass(hs_hbm, src_hbm, dst_hbm, zeros_hbm,
             aggp_hbm,
             acc_sh, sidx_v, didx_v, rows_v, sem):
    cid = lax.axis_index("c")
    sid = lax.axis_index("s")
    wid = sid * NC + cid

    @pl.when(sid == 0)
    def _():
        pltpu.sync_copy(zeros_hbm, acc_sh)

    pltpu.sync_copy(src_hbm.at[wid], sidx_v)
    pltpu.sync_copy(dst_hbm.at[wid], didx_v)
    plsc.subcore_barrier()

    def body(k, carry):
        pltpu.async_copy(hs_hbm.at[sidx_v.at[k]], rows_v, sem).wait()
        pltpu.sync_copy(rows_v, acc_sh.at[didx_v.at[k]], add=True)
        return carry

    lax.fori_loop(0, ECH, body, 0)
    plsc.subcore_barrier()

    for k in range(5):
        b = sid + k * NS

        @pl.when(b < NB)
        def _():
            pltpu.sync_copy(acc_sh.at[pl.ds(b * 128, 128)],
                            aggp_hbm.at[cid, pl.ds(b * 128, 128)])


def _col(lane2d):
    return lax.transpose(lane2d.reshape(NB, 1, 128), (0, 2, 1)).reshape(NP, 1)


def _tc_a_body(xg_ref, nt_ref, te_ref, degp_ref, w1_ref, hs_ref, dinv_ref):
    nt = _col(nt_ref[...])
    te = te_ref[...]
    x = xg_ref[...] + jnp.where(nt == 0, te[0:1],
                                jnp.where(nt == 1, te[1:2], te[2:3]))
    deg = _col(jnp.sum(degp_ref[...], axis=0)) + 1.0
    dinv = lax.rsqrt(deg)
    h = lax.dot_general(x, w1_ref[...], (((1,), (0,)), ((), ())),
                        precision=lax.Precision.HIGHEST)
    hs_ref[...] = dinv * h
    dinv_ref[...] = dinv


def _tc_c_body(aggp_ref, hs_ref, dinv_ref, b1_ref, w2_ref, out_ref):
    dinv = dinv_ref[...]
    a = aggp_ref[0] + aggp_ref[1] + hs_ref[...]
    z = jnp.maximum(dinv * a + b1_ref[...], 0.0)
    h2 = lax.dot_general(z, w2_ref[...], (((1,), (0,)), ((), ())),
                         precision=lax.Precision.HIGHEST)
    out_ref[...] = dinv * h2


def _tc_e_body(aggp_ref, hs_ref, dinv_ref, b2_ref, wc_ref, bc_ref, out_ref):
    a = aggp_ref[0] + aggp_ref[1] + hs_ref[...]
    z = dinv_ref[...] * a + b2_ref[...]
    out_ref[...] = lax.dot_general(z, wc_ref[...], (((1,), (0,)), ((), ())),
                                   precision=lax.Precision.HIGHEST) + bc_ref[...]


_tc_a = pl.pallas_call(
    _tc_a_body,
    out_shape=[
        jax.ShapeDtypeStruct((NP, D), jnp.float32),
        jax.ShapeDtypeStruct((NP, 1), jnp.float32),
    ],
)

_tc_c = pl.pallas_call(
    _tc_c_body,
    out_shape=jax.ShapeDtypeStruct((NP, D), jnp.float32),
)

_tc_e = pl.pallas_call(
    _tc_e_body,
    out_shape=jax.ShapeDtypeStruct((NP, D), jnp.float32),
)


def kernel(entity, node_type, edge_index, entity_emb, type_emb,
           W1, b1, W2, b2, Wc, bc):
    i32 = jnp.int32
    ent_p = jnp.concatenate(
        [entity.astype(i32), jnp.zeros((NPAD,), i32)])
    nt_lane = jnp.concatenate(
        [node_type.astype(i32), jnp.zeros((NPAD,), i32)]).reshape(NB, 128)
    pad_idx = (jnp.arange(EP - E, dtype=i32) % NPAD) + N
    src_p = jnp.concatenate([edge_index[0].astype(i32), pad_idx])
    dst_p = jnp.concatenate([edge_index[1].astype(i32), pad_idx])
    src3 = src_p.reshape(NW, ECH, 128)
    dst3 = dst_p.reshape(NW, ECH, 128)

    te_p = jnp.zeros((8, D), jnp.float32).at[:3].set(type_emb)
    wc_p = jnp.zeros((D, D), jnp.float32).at[:, :3].set(Wc)
    bc_p = jnp.zeros((1, D), jnp.float32).at[0, :3].set(bc)
    zcol = jnp.zeros((NB, 128), jnp.float32)
    zbig = jnp.zeros((NP, D), jnp.float32)

    xg, degp = _stage0(entity_emb, ent_p, dst_p, zcol)
    hs1, dinv = _tc_a(xg, nt_lane, te_p, degp, W1)
    agg1 = _msgpass(hs1, src3, dst3, zbig)
    hs2 = _tc_c(agg1, hs1, dinv, b1.reshape(1, D), W2)
    agg2 = _msgpass(hs2, src3, dst3, zbig)
    logits_p = _tc_e(agg2, hs2, dinv, b2.reshape(1, D), wc_p, bc_p)
    return logits_p[:N, :3]


# speedup vs baseline: 25.7331x; 1.2586x over previous
import functools

import jax
import jax.numpy as jnp
from jax import lax
from jax.experimental import pallas as pl
from jax.experimental.pallas import tpu as pltpu
from jax.experimental.pallas import tpu_sc as plsc

N = 10000
E = 320000
D = 128
NB = 79
NP = NB * 128
NPAD = NP - N
NC, NS = 2, 16
NW = NC * NS
CH = 128
NCH = 79
EPT = NCH * CH
EP = NW * EPT

_MESH = plsc.VectorSubcoreMesh(
    core_axis_name="c", subcore_axis_name="s", num_cores=NC, num_subcores=NS)


@functools.partial(
    pl.kernel,
    out_type=(
        jax.ShapeDtypeStruct((NP, D), jnp.float32),
        jax.ShapeDtypeStruct((NW, NB, 128), jnp.float32),
    ),
    mesh=_MESH,
    scratch_types=[
        pltpu.VMEM((128,), jnp.int32),
        pltpu.VMEM((128, D), jnp.float32),
        pltpu.VMEM((EPT,), jnp.int32),
        pltpu.VMEM((NB, 128), jnp.float32),
        pltpu.SemaphoreType.DMA,
    ],
    compiler_params=pltpu.CompilerParams(needs_layout_passes=False),
)
def _stage0(table_hbm, ent_hbm, dst_hbm, zcol_hbm,
            xg_hbm, degp_hbm,
            idx_v, rows_v, ddst_v, deg_v, sem):
    cid = lax.axis_index("c")
    sid = lax.axis_index("s")
    wid = sid * NC + cid

    for k in range(3):
        b = wid + k * NW

        @pl.when(b < NB)
        def _():
            pltpu.sync_copy(ent_hbm.at[pl.ds(b * 128, 128)], idx_v)
            pltpu.async_copy(table_hbm.at[idx_v], rows_v, sem).wait()
            pltpu.sync_copy(rows_v, xg_hbm.at[pl.ds(b * 128, 128), :])

    pltpu.sync_copy(zcol_hbm, deg_v)
    pltpu.sync_copy(dst_hbm.at[pl.ds(wid * EPT, EPT)], ddst_v)
    ones16 = jnp.full((16,), 1.0, jnp.float32)

    def body(i, carry):
        v = ddst_v[pl.ds(i * 16, 16)]
        plsc.addupdate_scatter(
            deg_v, [lax.shift_right_logical(v, 7), lax.bitwise_and(v, 127)],
            ones16)
        return carry

    lax.fori_loop(0, EPT // 16, body, 0)
    pltpu.sync_copy(deg_v, degp_hbm.at[wid])


@functools.partial(
    pl.kernel,
    out_type=jax.ShapeDtypeStruct((NC, NP, D), jnp.float32),
    mesh=_MESH,
    scratch_types=[
        pltpu.VMEM_SHARED((NP, D), jnp.float32),
        pltpu.VMEM((NCH, CH), jnp.int32),
        pltpu.VMEM((1, CH), jnp.int32),
        pltpu.VMEM((1, CH), jnp.int32),
        pltpu.VMEM((CH, D), jnp.float32),
        pltpu.VMEM((CH, D), jnp.float32),
        pltpu.SemaphoreType.DMA,
        pltpu.SemaphoreType.DMA,
        pltpu.SemaphoreType.DMA,
        pltpu.SemaphoreType.DMA,
    ],
)
def _msgpass(hs_hbm, src_hbm, dst_hbm, zeros_hbm,
             aggp_hbm,
             acc_sh, sidx_v, didx_a, didx_b, rows_a, rows_b,
             sem_a, sem_b, sem_da, sem_db):
    cid = lax.axis_index("c")
    sid = lax.axis_index("s")
    wid = sid * NC + cid

    @pl.when(sid == 0)
    def _():
        pltpu.sync_copy(zeros_hbm, acc_sh)

    pltpu.sync_copy(src_hbm.at[wid], sidx_v)
    plsc.subcore_barrier()

    pltpu.async_copy(dst_hbm.at[wid, pl.ds(0, 1)], didx_a, sem_da)
    pltpu.async_copy(dst_hbm.at[wid, pl.ds(1, 1)], didx_b, sem_db)
    pltpu.async_copy(hs_hbm.at[sidx_v.at[0]], rows_a, sem_a)

    def body(j, carry):
        a = 2 * j
        pltpu.make_async_copy(hs_hbm.at[sidx_v.at[a]], rows_a, sem_a).wait()
        pltpu.async_copy(hs_hbm.at[sidx_v.at[a + 1]], rows_b, sem_b)
        pltpu.make_async_copy(
            dst_hbm.at[wid, pl.ds(a, 1)], didx_a, sem_da).wait()
        pltpu.sync_copy(rows_a, acc_sh.at[didx_a.at[0]], add=True)

        @pl.when(a + 2 < NCH)
        def _():
            pltpu.async_copy(dst_hbm.at[wid, pl.ds(a + 2, 1)], didx_a, sem_da)

        pltpu.make_async_copy(
            hs_hbm.at[sidx_v.at[a + 1]], rows_b, sem_b).wait()

        @pl.when(a + 2 < NCH)
        def _():
            pltpu.async_copy(hs_hbm.at[sidx_v.at[a + 2]], rows_a, sem_a)

        pltpu.make_async_copy(
            dst_hbm.at[wid, pl.ds(a + 1, 1)], didx_b, sem_db).wait()
        pltpu.sync_copy(rows_b, acc_sh.at[didx_b.at[0]], add=True)

        @pl.when(a + 3 < NCH)
        def _():
            pltpu.async_copy(dst_hbm.at[wid, pl.ds(a + 3, 1)], didx_b, sem_db)

        return carry

    lax.fori_loop(0, NCH // 2, body, 0)
    pltpu.make_async_copy(
        hs_hbm.at[sidx_v.at[NCH - 1]], rows_a, sem_a).wait()
    pltpu.make_async_copy(
        dst_hbm.at[wid, pl.ds(NCH - 1, 1)], didx_a, sem_da).wait()
    pltpu.sync_copy(rows_a, acc_sh.at[didx_a.at[0]], add=True)
    plsc.subcore_barrier()

    for k in range(5):
        b = sid + k * NS

        @pl.when(b < NB)
        def _():
            pltpu.sync_copy(acc_sh.at[pl.ds(b * 128, 128)],
                            aggp_hbm.at[cid, pl.ds(b * 128, 128)])


def _col(lane2d):
    return lax.transpose(lane2d.reshape(NB, 1, 128), (0, 2, 1)).reshape(NP, 1)


def _tc_a_body(xg_ref, nt_ref, te_ref, degp_ref, w1_ref, hs_ref, dinv_ref):
    nt = _col(nt_ref[...])
    te = te_ref[...]
    x = xg_ref[...] + jnp.where(nt == 0, te[0:1],
                                jnp.where(nt == 1, te[1:2], te[2:3]))
    deg = _col(jnp.sum(degp_ref[...], axis=0)) + 1.0
    dinv = lax.rsqrt(deg)
    h = lax.dot_general(x, w1_ref[...], (((1,), (0,)), ((), ())),
                        precision=lax.Precision.HIGHEST)
    hs_ref[...] = dinv * h
    dinv_ref[...] = dinv


def _tc_c_body(aggp_ref, hs_ref, dinv_ref, b1_ref, w2_ref, out_ref):
    dinv = dinv_ref[...]
    a = aggp_ref[0] + aggp_ref[1] + hs_ref[...]
    z = jnp.maximum(dinv * a + b1_ref[...], 0.0)
    h2 = lax.dot_general(z, w2_ref[...], (((1,), (0,)), ((), ())),
                         precision=lax.Precision.HIGHEST)
    out_ref[...] = dinv * h2


def _tc_e_body(aggp_ref, hs_ref, dinv_ref, b2_ref, wc_ref, bc_ref, out_ref):
    a = aggp_ref[0] + aggp_ref[1] + hs_ref[...]
    z = dinv_ref[...] * a + b2_ref[...]
    out_ref[...] = lax.dot_general(z, wc_ref[...], (((1,), (0,)), ((), ())),
                                   precision=lax.Precision.HIGHEST) + bc_ref[...]


_tc_a = pl.pallas_call(
    _tc_a_body,
    out_shape=[
        jax.ShapeDtypeStruct((NP, D), jnp.float32),
        jax.ShapeDtypeStruct((NP, 1), jnp.float32),
    ],
)

_tc_c = pl.pallas_call(
    _tc_c_body,
    out_shape=jax.ShapeDtypeStruct((NP, D), jnp.float32),
)

_tc_e = pl.pallas_call(
    _tc_e_body,
    out_shape=jax.ShapeDtypeStruct((NP, D), jnp.float32),
)


def kernel(entity, node_type, edge_index, entity_emb, type_emb,
           W1, b1, W2, b2, Wc, bc):
    i32 = jnp.int32
    ent_p = jnp.concatenate(
        [entity.astype(i32), jnp.zeros((NPAD,), i32)])
    nt_lane = jnp.concatenate(
        [node_type.astype(i32), jnp.zeros((NPAD,), i32)]).reshape(NB, 128)
    pad_idx = (jnp.arange(EP - E, dtype=i32) % NPAD) + N
    src_p = jnp.concatenate([edge_index[0].astype(i32), pad_idx])
    dst_p = jnp.concatenate([edge_index[1].astype(i32), pad_idx])
    src3 = src_p.reshape(NW, NCH, CH)
    dst3 = dst_p.reshape(NW, NCH, CH)

    te_p = jnp.zeros((8, D), jnp.float32).at[:3].set(type_emb)
    wc_p = jnp.zeros((D, D), jnp.float32).at[:, :3].set(Wc)
    bc_p = jnp.zeros((1, D), jnp.float32).at[0, :3].set(bc)
    zcol = jnp.zeros((NB, 128), jnp.float32)
    zbig = jnp.zeros((NP, D), jnp.float32)

    xg, degp = _stage0(entity_emb, ent_p, dst_p, zcol)
    hs1, dinv = _tc_a(xg, nt_lane, te_p, degp, W1)
    agg1 = _msgpass(hs1, src3, dst3, zbig)
    hs2 = _tc_c(agg1, hs1, dinv, b1.reshape(1, D), W2)
    agg2 = _msgpass(hs2, src3, dst3, zbig)
    logits_p = _tc_e(agg2, hs2, dinv, b2.reshape(1, D), wc_p, bc_p)
    return logits_p[:N, :3]
